# Initial kernel scaffold; baseline (speedup 1.0000x reference)
import jax
import jax.numpy as jnp
from jax.experimental import pallas as pl


def kernel(x0, x1, W0, b0, W1, b1, a_src, a_dst, Wout, bout, edge_index):
    raise NotImplementedError("write your pallas kernel here")


# speedup vs baseline: 1.1795x; 1.1795x over previous
import jax
import jax.numpy as jnp
from jax.experimental import pallas as pl

ALPHA = 0.2
N_NODES = 10000
DH = 512
NH = 4


def _leaky(x):
    return jnp.where(x >= 0, x, ALPHA * x)


def _proj_kernel(nb_half, x_ref, W_ref, b_ref, o_ref):
    t = pl.program_id(0) // nb_half
    o_ref[...] = _leaky(
        jnp.dot(x_ref[...], W_ref[t], preferred_element_type=jnp.float32)
        + b_ref[t]
    )


def kernel(x0, x1, W0, b0, W1, b1, a_src, a_dst, Wout, bout, edge_index):
    x = jnp.concatenate([x0, x1], axis=0)
    Ws = jnp.stack([W0, W1])
    bs = jnp.stack([b0, b1])
    BLK = 1000
    NB = x.shape[0] // BLK
    import functools
    h = pl.pallas_call(
        functools.partial(_proj_kernel, NB // 2),
        grid=(NB,),
        in_specs=[
            pl.BlockSpec((BLK, x.shape[1]), lambda i: (i, 0)),
            pl.BlockSpec((2, x.shape[1], DH), lambda i: (0, 0, 0)),
            pl.BlockSpec((2, DH), lambda i: (0, 0)),
        ],
        out_specs=pl.BlockSpec((BLK, DH), lambda i: (i, 0)),
        out_shape=jax.ShapeDtypeStruct((x.shape[0], DH), jnp.float32),
    )(x, Ws, bs)

    src, dst = edge_index[0], edge_index[1]
    s = h @ a_src.T
    d = h @ a_dst.T
    e = _leaky(jnp.take(s, src, axis=0) + jnp.take(d, dst, axis=0))
    ex = jnp.exp(e)
    denom = jax.ops.segment_sum(ex, dst, num_segments=N_NODES)
    hs = jnp.take(h, src, axis=0)
    feats = []
    for k in range(NH):
        m = jax.ops.segment_max(ex[:, k:k + 1] * hs, dst, num_segments=N_NODES)
        m = jnp.where(jnp.isfinite(m), m, 0.0)
        feats.append(m / (denom[:, k:k + 1] + 1e-16))
    feat = sum(feats) / NH
    out = feat @ Wout + bout
    nrm = jnp.sqrt(jnp.sum(out * out, axis=1, keepdims=True))
    return out / jnp.maximum(nrm, 1e-12)
